# baseline (speedup 1.0000x reference)
import functools

import jax
import jax.numpy as jnp
from jax import lax
from jax.experimental import pallas as pl
from jax.experimental.pallas import tpu as pltpu
from jax.experimental.pallas import tpu_sc as plsc

N_NODES = 10000
D = 128
NC = 2
NS = 16
NW = NC * NS
CHUNK = 128
NP = 10240
ROWS_PER_TILE = NP // NS
DEG_W = 16

_ROW_CHUNKS = [(k * CHUNK, CHUNK) for k in range(ROWS_PER_TILE // CHUNK)]
if ROWS_PER_TILE % CHUNK:
    _ROW_CHUNKS.append((ROWS_PER_TILE - ROWS_PER_TILE % CHUNK, ROWS_PER_TILE % CHUNK))

_MESH = plsc.VectorSubcoreMesh(core_axis_name="c", subcore_axis_name="s")


def _sc_deg(dst_pad, n_chunks):
    @functools.partial(
        pl.kernel,
        out_type=jax.ShapeDtypeStruct((NC, NP, DEG_W), jnp.float32),
        mesh=_MESH,
        scratch_types=[
            pltpu.VMEM((CHUNK,), jnp.int32),
            pltpu.VMEM((CHUNK, DEG_W), jnp.float32),
            pltpu.VMEM((CHUNK, DEG_W), jnp.float32),
            pltpu.VMEM_SHARED((NP, DEG_W), jnp.float32),
        ],
    )
    def deg_kernel(dst_hbm, out_hbm, idx_v, ones_v, zeros_v, hist_sh):
        cid = lax.axis_index("c")
        sid = lax.axis_index("s")
        wid = cid * NS + sid

        def fill(r, _):
            ones_v[r, :] = jnp.full((DEG_W,), 1.0, jnp.float32)
            zeros_v[r, :] = jnp.zeros((DEG_W,), jnp.float32)
            return 0

        lax.fori_loop(0, CHUNK, fill, 0)
        for off, sz in _ROW_CHUNKS:
            pltpu.sync_copy(zeros_v.at[pl.ds(0, sz)],
                            hist_sh.at[pl.ds(sid * ROWS_PER_TILE + off, sz)])
        plsc.subcore_barrier()

        base = wid * n_chunks * CHUNK

        def body(c, _):
            pltpu.sync_copy(dst_hbm.at[pl.ds(base + c * CHUNK, CHUNK)], idx_v)
            pltpu.sync_copy(ones_v, hist_sh.at[idx_v], add=True)
            return 0

        lax.fori_loop(0, n_chunks, body, 0)
        plsc.subcore_barrier()
        for off, sz in _ROW_CHUNKS:
            r0 = pl.multiple_of(sid * ROWS_PER_TILE + off, 32)
            pltpu.sync_copy(hist_sh.at[pl.ds(r0, sz)], out_hbm.at[cid, pl.ds(r0, sz)])

    return deg_kernel(dst_pad)


@functools.lru_cache(maxsize=None)
def _make_sc_agg(n_chunks):
    assert n_chunks % 2 == 0
    npairs = n_chunks // 2

    @functools.partial(
        pl.kernel,
        out_type=jax.ShapeDtypeStruct((NC, NP, D), jnp.float32),
        mesh=_MESH,
        scratch_types=[
            pltpu.VMEM((CHUNK,), jnp.int32),
            pltpu.VMEM((CHUNK,), jnp.int32),
            pltpu.VMEM((CHUNK,), jnp.int32),
            pltpu.VMEM((CHUNK,), jnp.int32),
            pltpu.VMEM((CHUNK, D), jnp.float32),
            pltpu.VMEM((CHUNK, D), jnp.float32),
            pltpu.VMEM_SHARED((NP, D), jnp.float32),
            [pltpu.SemaphoreType.DMA] * 2,
        ],
    )
    def agg_kernel(x_hbm, src_hbm, dst_hbm, out_hbm, siA, diA, siB, diB,
                   rA, rB, acc_sh, sems):
        gA, gB = sems
        cid = lax.axis_index("c")
        sid = lax.axis_index("s")
        wid = cid * NS + sid

        def fill(r, _):
            def fill_c(c, _):
                rA[r, pl.ds(c * 16, 16)] = jnp.zeros((16,), jnp.float32)
                return 0

            lax.fori_loop(0, D // 16, fill_c, 0)
            return 0

        lax.fori_loop(0, CHUNK, fill, 0)
        for off, sz in _ROW_CHUNKS:
            pltpu.sync_copy(rA.at[pl.ds(0, sz)],
                            acc_sh.at[pl.ds(sid * ROWS_PER_TILE + off, sz)])
        plsc.subcore_barrier()

        base = wid * n_chunks * CHUNK

        def pair_body(p, _):
            e0 = base + 2 * p * CHUNK
            pltpu.sync_copy(src_hbm.at[pl.ds(e0, CHUNK)], siA)
            pltpu.sync_copy(dst_hbm.at[pl.ds(e0, CHUNK)], diA)
            pltpu.sync_copy(src_hbm.at[pl.ds(e0 + CHUNK, CHUNK)], siB)
            pltpu.sync_copy(dst_hbm.at[pl.ds(e0 + CHUNK, CHUNK)], diB)
            dA = pltpu.async_copy(x_hbm.at[siA], rA, gA)
            dB = pltpu.async_copy(x_hbm.at[siB], rB, gB)
            dA.wait()
            pltpu.sync_copy(rA, acc_sh.at[diA], add=True)
            dB.wait()
            pltpu.sync_copy(rB, acc_sh.at[diB], add=True)
            return 0

        lax.fori_loop(0, npairs, pair_body, 0)
        plsc.subcore_barrier()
        for off, sz in _ROW_CHUNKS:
            r0 = pl.multiple_of(sid * ROWS_PER_TILE + off, 32)
            pltpu.sync_copy(acc_sh.at[pl.ds(r0, sz)], out_hbm.at[cid, pl.ds(r0, sz)])

    return agg_kernel


_BR = 1024


def _dinv_block(degp_blk):
    return lax.rsqrt(degp_blk[0, :, 0:1] + degp_blk[1, :, 0:1] + 1.0)


def _tc_prep(degp, x_pad):
    def body(degp_ref, x_ref, xs_ref):
        xs_ref[...] = x_ref[...] * _dinv_block(degp_ref[...])

    return pl.pallas_call(
        body,
        grid=(NP // _BR,),
        in_specs=[
            pl.BlockSpec((NC, _BR, DEG_W), lambda i: (0, i, 0)),
            pl.BlockSpec((_BR, D), lambda i: (i, 0)),
        ],
        out_specs=pl.BlockSpec((_BR, D), lambda i: (i, 0)),
        out_shape=jax.ShapeDtypeStruct((NP, D), jnp.float32),
    )(degp, x_pad)


def _tc_mid(p, xs, degp, W1, b1, W2):
    def body(p_ref, xs_ref, degp_ref, w1_ref, b1_ref, w2_ref, hs_ref):
        i = pl.program_id(0)
        dinv = _dinv_block(degp_ref[...])
        y = (p_ref[0] + p_ref[1] + xs_ref[...]) * dinv
        x1 = jnp.maximum(jnp.dot(y, w1_ref[...], preferred_element_type=jnp.float32) + b1_ref[...], 0.0)
        h = jnp.dot(x1, w2_ref[...], preferred_element_type=jnp.float32)
        row = i * _BR + lax.broadcasted_iota(jnp.int32, (_BR, 1), 0)
        hs_ref[...] = jnp.where(row < N_NODES, h * dinv, 0.0)

    return pl.pallas_call(
        body,
        grid=(NP // _BR,),
        in_specs=[
            pl.BlockSpec((NC, _BR, D), lambda i: (0, i, 0)),
            pl.BlockSpec((_BR, D), lambda i: (i, 0)),
            pl.BlockSpec((NC, _BR, DEG_W), lambda i: (0, i, 0)),
            pl.BlockSpec((D, 2 * D), lambda i: (0, 0)),
            pl.BlockSpec((1, 2 * D), lambda i: (0, 0)),
            pl.BlockSpec((2 * D, D), lambda i: (0, 0)),
        ],
        out_specs=pl.BlockSpec((_BR, D), lambda i: (i, 0)),
        out_shape=jax.ShapeDtypeStruct((NP, D), jnp.float32),
    )(p, xs, degp, W1, b1, W2)


def _tc_final(q, hs, degp, b2):
    def body(q_ref, hs_ref, degp_ref, b2_ref, out_ref):
        dinv = _dinv_block(degp_ref[...])
        out_ref[...] = (q_ref[0] + q_ref[1] + hs_ref[...]) * dinv + b2_ref[...]

    return pl.pallas_call(
        body,
        grid=(NP // _BR,),
        in_specs=[
            pl.BlockSpec((NC, _BR, D), lambda i: (0, i, 0)),
            pl.BlockSpec((_BR, D), lambda i: (i, 0)),
            pl.BlockSpec((NC, _BR, DEG_W), lambda i: (0, i, 0)),
            pl.BlockSpec((1, D), lambda i: (0, 0)),
        ],
        out_specs=pl.BlockSpec((_BR, D), lambda i: (i, 0)),
        out_shape=jax.ShapeDtypeStruct((NP, D), jnp.float32),
    )(q, hs, degp, b2)


def kernel(sent_encoder_embeds, edge_index, W1, b1, W2, b2):
    x = sent_encoder_embeds
    src = edge_index[0]
    dst = edge_index[1]
    n_edges = src.shape[0]
    per_pair = NW * CHUNK * 2
    ep = ((n_edges + per_pair - 1) // per_pair) * per_pair
    n_chunks = ep // (NW * CHUNK)
    pad = jnp.full((ep - n_edges,), N_NODES, jnp.int32)
    src_p = jnp.concatenate([src, pad])
    dst_p = jnp.concatenate([dst, pad])
    x_pad = jnp.pad(x, ((0, NP - N_NODES), (0, 0)))

    sc_agg = _make_sc_agg(n_chunks)
    degc = _sc_deg(dst_p, n_chunks)
    xs = _tc_prep(degc, x_pad)
    p = sc_agg(xs, src_p, dst_p)
    hs = _tc_mid(p, xs, degc, W1.astype(jnp.float32), b1.reshape(1, -1), W2.astype(jnp.float32))
    q = sc_agg(hs, src_p, dst_p)
    out = _tc_final(q, hs, degc, b2.reshape(1, -1))
    return out[:N_NODES]


# speedup vs baseline: 1.1137x; 1.1137x over previous
import functools

import jax
import jax.numpy as jnp
from jax import lax
from jax.experimental import pallas as pl
from jax.experimental.pallas import tpu as pltpu
from jax.experimental.pallas import tpu_sc as plsc

N_NODES = 10000
D = 128
NC = 2
NS = 16
NW = NC * NS
CHUNK = 128
NP = 10240
ROWS_PER_TILE = NP // NS
DEG_W = 16
SEG = 16

_ROW_CHUNKS = [(k * CHUNK, CHUNK) for k in range(ROWS_PER_TILE // CHUNK)]
if ROWS_PER_TILE % CHUNK:
    _ROW_CHUNKS.append((ROWS_PER_TILE - ROWS_PER_TILE % CHUNK, ROWS_PER_TILE % CHUNK))

_MESH = plsc.VectorSubcoreMesh(core_axis_name="c", subcore_axis_name="s")


def _sc_deg(dst_pad, n_chunks):
    @functools.partial(
        pl.kernel,
        out_type=jax.ShapeDtypeStruct((NC, NP, DEG_W), jnp.float32),
        mesh=_MESH,
        scratch_types=[
            pltpu.VMEM((CHUNK,), jnp.int32),
            pltpu.VMEM((CHUNK, DEG_W), jnp.float32),
            pltpu.VMEM((CHUNK, DEG_W), jnp.float32),
            pltpu.VMEM_SHARED((NP, DEG_W), jnp.float32),
        ],
    )
    def deg_kernel(dst_hbm, out_hbm, idx_v, ones_v, zeros_v, hist_sh):
        cid = lax.axis_index("c")
        sid = lax.axis_index("s")
        wid = cid * NS + sid

        def fill(r, _):
            ones_v[r, :] = jnp.full((DEG_W,), 1.0, jnp.float32)
            zeros_v[r, :] = jnp.zeros((DEG_W,), jnp.float32)
            return 0

        lax.fori_loop(0, CHUNK, fill, 0)
        for off, sz in _ROW_CHUNKS:
            pltpu.sync_copy(zeros_v.at[pl.ds(0, sz)],
                            hist_sh.at[pl.ds(sid * ROWS_PER_TILE + off, sz)])
        plsc.subcore_barrier()

        base = wid * n_chunks * CHUNK

        def body(c, _):
            pltpu.sync_copy(dst_hbm.at[pl.ds(base + c * CHUNK, CHUNK)], idx_v)
            pltpu.sync_copy(ones_v, hist_sh.at[idx_v], add=True)
            return 0

        lax.fori_loop(0, n_chunks, body, 0)
        plsc.subcore_barrier()
        for off, sz in _ROW_CHUNKS:
            r0 = pl.multiple_of(sid * ROWS_PER_TILE + off, 32)
            pltpu.sync_copy(hist_sh.at[pl.ds(r0, sz)], out_hbm.at[cid, pl.ds(r0, sz)])

    return deg_kernel(dst_pad)


@functools.lru_cache(maxsize=None)
def _make_sc_agg(n_chunks):
    assert n_chunks % 2 == 0

    @functools.partial(
        pl.kernel,
        out_type=jax.ShapeDtypeStruct((NC, NP, D), jnp.float32),
        mesh=_MESH,
        scratch_types=[
            pltpu.VMEM((CHUNK,), jnp.int32),
            pltpu.VMEM((CHUNK,), jnp.int32),
            pltpu.VMEM((CHUNK,), jnp.int32),
            pltpu.VMEM((CHUNK,), jnp.int32),
            pltpu.VMEM((CHUNK, D), jnp.float32),
            pltpu.VMEM((CHUNK, D), jnp.float32),
            pltpu.VMEM_SHARED((NP, D), jnp.float32),
            [pltpu.SemaphoreType.DMA] * 6,
        ],
    )
    def agg_kernel(x_hbm, src_hbm, dst_hbm, out_hbm, siA, diA, siB, diB,
                   rA, rB, acc_sh, sems):
        gA, gB, isA, idA, isB, idB = sems
        cid = lax.axis_index("c")
        sid = lax.axis_index("s")
        wid = cid * NS + sid

        def fill(r, _):
            def fill_c(c, _):
                rA[r, pl.ds(c * 16, 16)] = jnp.zeros((16,), jnp.float32)
                return 0

            lax.fori_loop(0, D // 16, fill_c, 0)
            return 0

        lax.fori_loop(0, CHUNK, fill, 0)
        for off, sz in _ROW_CHUNKS:
            pltpu.sync_copy(rA.at[pl.ds(0, sz)],
                            acc_sh.at[pl.ds(sid * ROWS_PER_TILE + off, sz)])
        plsc.subcore_barrier()

        base = wid * n_chunks * CHUNK

        def load_idx(c, si, di, ssem, dsem):
            e0 = pl.multiple_of(base + c * CHUNK, 128)
            ds_ = pltpu.async_copy(src_hbm.at[pl.ds(e0, CHUNK)], si, ssem)
            dd_ = pltpu.async_copy(dst_hbm.at[pl.ds(e0, CHUNK)], di, dsem)
            return ds_, dd_

        def pair_body(p, prefetch_next):
            c0 = 2 * p
            dA = pltpu.async_copy(x_hbm.at[siA], rA, gA)
            dsB, ddB = load_idx(c0 + 1, siB, diB, isB, idB)
            dsB.wait()
            ddB.wait()
            dA.wait()
            dB = pltpu.async_copy(x_hbm.at[siB], rB, gB)
            pltpu.sync_copy(rA, acc_sh.at[diA], add=True)
            if prefetch_next:
                dsA, ddA = load_idx(c0 + 2, siA, diA, isA, idA)
            dB.wait()
            pltpu.sync_copy(rB, acc_sh.at[diB], add=True)
            if prefetch_next:
                dsA.wait()
                ddA.wait()
            return 0

        pltpu.sync_copy(src_hbm.at[pl.ds(base, CHUNK)], siA)
        pltpu.sync_copy(dst_hbm.at[pl.ds(base, CHUNK)], diA)
        npairs = n_chunks // 2

        def body(p, _):
            return pair_body(p, True)

        lax.fori_loop(0, npairs - 1, body, 0)
        pair_body(npairs - 1, False)
        plsc.subcore_barrier()
        for off, sz in _ROW_CHUNKS:
            r0 = pl.multiple_of(sid * ROWS_PER_TILE + off, 32)
            pltpu.sync_copy(acc_sh.at[pl.ds(r0, sz)], out_hbm.at[cid, pl.ds(r0, sz)])

    return agg_kernel


_BR = 1024


def _dinv_block(degp_blk):
    return lax.rsqrt(degp_blk[0, :, 0:1] + degp_blk[1, :, 0:1] + 1.0)


def _tc_prep(degp, x_pad):
    def body(degp_ref, x_ref, xs_ref):
        xs_ref[...] = x_ref[...] * _dinv_block(degp_ref[...])

    return pl.pallas_call(
        body,
        grid=(NP // _BR,),
        in_specs=[
            pl.BlockSpec((NC, _BR, DEG_W), lambda i: (0, i, 0)),
            pl.BlockSpec((_BR, D), lambda i: (i, 0)),
        ],
        out_specs=pl.BlockSpec((_BR, D), lambda i: (i, 0)),
        out_shape=jax.ShapeDtypeStruct((NP, D), jnp.float32),
    )(degp, x_pad)


def _tc_mid(p, xs, degp, W1, b1, W2):
    def body(p_ref, xs_ref, degp_ref, w1_ref, b1_ref, w2_ref, hs_ref):
        i = pl.program_id(0)
        dinv = _dinv_block(degp_ref[...])
        y = (p_ref[0] + p_ref[1] + xs_ref[...]) * dinv
        x1 = jnp.maximum(jnp.dot(y, w1_ref[...], preferred_element_type=jnp.float32) + b1_ref[...], 0.0)
        h = jnp.dot(x1, w2_ref[...], preferred_element_type=jnp.float32)
        row = i * _BR + lax.broadcasted_iota(jnp.int32, (_BR, 1), 0)
        hs_ref[...] = jnp.where(row < N_NODES, h * dinv, 0.0)

    return pl.pallas_call(
        body,
        grid=(NP // _BR,),
        in_specs=[
            pl.BlockSpec((NC, _BR, D), lambda i: (0, i, 0)),
            pl.BlockSpec((_BR, D), lambda i: (i, 0)),
            pl.BlockSpec((NC, _BR, DEG_W), lambda i: (0, i, 0)),
            pl.BlockSpec((D, 2 * D), lambda i: (0, 0)),
            pl.BlockSpec((1, 2 * D), lambda i: (0, 0)),
            pl.BlockSpec((2 * D, D), lambda i: (0, 0)),
        ],
        out_specs=pl.BlockSpec((_BR, D), lambda i: (i, 0)),
        out_shape=jax.ShapeDtypeStruct((NP, D), jnp.float32),
    )(p, xs, degp, W1, b1, W2)


def _tc_final(q, hs, degp, b2):
    def body(q_ref, hs_ref, degp_ref, b2_ref, out_ref):
        dinv = _dinv_block(degp_ref[...])
        out_ref[...] = (q_ref[0] + q_ref[1] + hs_ref[...]) * dinv + b2_ref[...]

    return pl.pallas_call(
        body,
        grid=(NP // _BR,),
        in_specs=[
            pl.BlockSpec((NC, _BR, D), lambda i: (0, i, 0)),
            pl.BlockSpec((_BR, D), lambda i: (i, 0)),
            pl.BlockSpec((NC, _BR, DEG_W), lambda i: (0, i, 0)),
            pl.BlockSpec((1, D), lambda i: (0, 0)),
        ],
        out_specs=pl.BlockSpec((_BR, D), lambda i: (i, 0)),
        out_shape=jax.ShapeDtypeStruct((NP, D), jnp.float32),
    )(q, hs, degp, b2)


def kernel(sent_encoder_embeds, edge_index, W1, b1, W2, b2):
    x = sent_encoder_embeds
    src = edge_index[0]
    dst = edge_index[1]
    n_edges = src.shape[0]
    per_seg = NW * CHUNK * SEG
    ep = ((n_edges + per_seg - 1) // per_seg) * per_seg
    n_chunks = ep // (NW * CHUNK)
    pad = jnp.full((ep - n_edges,), N_NODES, jnp.int32)
    src_p = jnp.concatenate([src, pad])
    dst_p = jnp.concatenate([dst, pad])
    x_pad = jnp.pad(x, ((0, NP - N_NODES), (0, 0)))

    sc_agg = _make_sc_agg(n_chunks)
    degc = _sc_deg(dst_p, n_chunks)
    xs = _tc_prep(degc, x_pad)
    p = sc_agg(xs, src_p, dst_p)
    hs = _tc_mid(p, xs, degc, W1.astype(jnp.float32), b1.reshape(1, -1), W2.astype(jnp.float32))
    q = sc_agg(hs, src_p, dst_p)
    out = _tc_final(q, hs, degc, b2.reshape(1, -1))
    return out[:N_NODES]


# speedup vs baseline: 1.1154x; 1.0015x over previous
import functools

import jax
import jax.numpy as jnp
from jax import lax
from jax.experimental import pallas as pl
from jax.experimental.pallas import tpu as pltpu
from jax.experimental.pallas import tpu_sc as plsc

N_NODES = 10000
D = 128
NC = 2
NS = 16
NW = NC * NS
CHUNK = 128
NP = 10240
ROWS_PER_TILE = NP // NS
DEG_W = 16
SEG = 16

_ROW_CHUNKS = [(k * CHUNK, CHUNK) for k in range(ROWS_PER_TILE // CHUNK)]
if ROWS_PER_TILE % CHUNK:
    _ROW_CHUNKS.append((ROWS_PER_TILE - ROWS_PER_TILE % CHUNK, ROWS_PER_TILE % CHUNK))

_MESH = plsc.VectorSubcoreMesh(core_axis_name="c", subcore_axis_name="s")


def _sc_deg(dst_pad, n_chunks):
    @functools.partial(
        pl.kernel,
        out_type=jax.ShapeDtypeStruct((NC, NP, DEG_W), jnp.float32),
        mesh=_MESH,
        scratch_types=[
            pltpu.VMEM((CHUNK,), jnp.int32),
            pltpu.VMEM((CHUNK, DEG_W), jnp.float32),
            pltpu.VMEM((CHUNK, DEG_W), jnp.float32),
            pltpu.VMEM_SHARED((NP, DEG_W), jnp.float32),
        ],
    )
    def deg_kernel(dst_hbm, out_hbm, idx_v, ones_v, zeros_v, hist_sh):
        cid = lax.axis_index("c")
        sid = lax.axis_index("s")
        wid = cid * NS + sid

        def fill(r, _):
            ones_v[r, :] = jnp.full((DEG_W,), 1.0, jnp.float32)
            zeros_v[r, :] = jnp.zeros((DEG_W,), jnp.float32)
            return 0

        lax.fori_loop(0, CHUNK, fill, 0)
        for off, sz in _ROW_CHUNKS:
            pltpu.sync_copy(zeros_v.at[pl.ds(0, sz)],
                            hist_sh.at[pl.ds(sid * ROWS_PER_TILE + off, sz)])
        plsc.subcore_barrier()

        base = wid * n_chunks * CHUNK

        def body(c, _):
            pltpu.sync_copy(dst_hbm.at[pl.ds(base + c * CHUNK, CHUNK)], idx_v)
            pltpu.sync_copy(ones_v, hist_sh.at[idx_v], add=True)
            return 0

        lax.fori_loop(0, n_chunks, body, 0)
        plsc.subcore_barrier()
        for off, sz in _ROW_CHUNKS:
            r0 = pl.multiple_of(sid * ROWS_PER_TILE + off, 32)
            pltpu.sync_copy(hist_sh.at[pl.ds(r0, sz)], out_hbm.at[cid, pl.ds(r0, sz)])

    return deg_kernel(dst_pad)


@functools.lru_cache(maxsize=None)
def _make_sc_agg(n_chunks):
    assert n_chunks % 2 == 0

    @functools.partial(
        pl.kernel,
        out_type=jax.ShapeDtypeStruct((NC, NP, D), jnp.float32),
        mesh=_MESH,
        scratch_types=[
            pltpu.VMEM((CHUNK,), jnp.int32),
            pltpu.VMEM((CHUNK,), jnp.int32),
            pltpu.VMEM((CHUNK,), jnp.int32),
            pltpu.VMEM((CHUNK,), jnp.int32),
            pltpu.VMEM((CHUNK, D), jnp.float32),
            pltpu.VMEM((CHUNK, D), jnp.float32),
            pltpu.VMEM_SHARED((NP, D), jnp.float32),
            [pltpu.SemaphoreType.DMA] * 8,
        ],
    )
    def agg_kernel(x_hbm, src_hbm, dst_hbm, out_hbm, siA, diA, siB, diB,
                   rA, rB, acc_sh, sems):
        gA, gB, isA, idA, isB, idB, aA, aB = sems
        cid = lax.axis_index("c")
        sid = lax.axis_index("s")
        wid = cid * NS + sid

        def fill(r, _):
            def fill_c(c, _):
                rA[r, pl.ds(c * 16, 16)] = jnp.zeros((16,), jnp.float32)
                return 0

            lax.fori_loop(0, D // 16, fill_c, 0)
            return 0

        lax.fori_loop(0, CHUNK, fill, 0)
        for off, sz in _ROW_CHUNKS:
            pltpu.sync_copy(rA.at[pl.ds(0, sz)],
                            acc_sh.at[pl.ds(sid * ROWS_PER_TILE + off, sz)])
        plsc.subcore_barrier()

        base = wid * n_chunks * CHUNK

        def load_idx(c, si, di, ssem, dsem):
            e0 = pl.multiple_of(base + c * CHUNK, 128)
            ds_ = pltpu.async_copy(src_hbm.at[pl.ds(e0, CHUNK)], si, ssem)
            dd_ = pltpu.async_copy(dst_hbm.at[pl.ds(e0, CHUNK)], di, dsem)
            return ds_, dd_

        def pair_body(p, prefetch_next):
            c0 = 2 * p
            dA = pltpu.async_copy(x_hbm.at[siA], rA, gA)
            dsB, ddB = load_idx(c0 + 1, siB, diB, isB, idB)
            dsB.wait()
            ddB.wait()
            dA.wait()
            dB = pltpu.async_copy(x_hbm.at[siB], rB, gB)
            sA = pltpu.async_copy(rA, acc_sh.at[diA], aA, add=True)
            dB.wait()
            sB = pltpu.async_copy(rB, acc_sh.at[diB], aB, add=True)
            sA.wait()
            if prefetch_next:
                dsA, ddA = load_idx(c0 + 2, siA, diA, isA, idA)
            sB.wait()
            if prefetch_next:
                dsA.wait()
                ddA.wait()
            return 0

        pltpu.sync_copy(src_hbm.at[pl.ds(base, CHUNK)], siA)
        pltpu.sync_copy(dst_hbm.at[pl.ds(base, CHUNK)], diA)
        npairs = n_chunks // 2

        def body(p, _):
            return pair_body(p, True)

        lax.fori_loop(0, npairs - 1, body, 0)
        pair_body(npairs - 1, False)
        plsc.subcore_barrier()
        for off, sz in _ROW_CHUNKS:
            r0 = pl.multiple_of(sid * ROWS_PER_TILE + off, 32)
            pltpu.sync_copy(acc_sh.at[pl.ds(r0, sz)], out_hbm.at[cid, pl.ds(r0, sz)])

    return agg_kernel


_BR = 1024


def _dinv_block(degp_blk):
    return lax.rsqrt(degp_blk[0, :, 0:1] + degp_blk[1, :, 0:1] + 1.0)


def _tc_prep(degp, x_pad):
    def body(degp_ref, x_ref, xs_ref):
        xs_ref[...] = x_ref[...] * _dinv_block(degp_ref[...])

    return pl.pallas_call(
        body,
        grid=(NP // _BR,),
        in_specs=[
            pl.BlockSpec((NC, _BR, DEG_W), lambda i: (0, i, 0)),
            pl.BlockSpec((_BR, D), lambda i: (i, 0)),
        ],
        out_specs=pl.BlockSpec((_BR, D), lambda i: (i, 0)),
        out_shape=jax.ShapeDtypeStruct((NP, D), jnp.float32),
    )(degp, x_pad)


def _tc_mid(p, xs, degp, W1, b1, W2):
    def body(p_ref, xs_ref, degp_ref, w1_ref, b1_ref, w2_ref, hs_ref):
        i = pl.program_id(0)
        dinv = _dinv_block(degp_ref[...])
        y = (p_ref[0] + p_ref[1] + xs_ref[...]) * dinv
        x1 = jnp.maximum(jnp.dot(y, w1_ref[...], preferred_element_type=jnp.float32) + b1_ref[...], 0.0)
        h = jnp.dot(x1, w2_ref[...], preferred_element_type=jnp.float32)
        row = i * _BR + lax.broadcasted_iota(jnp.int32, (_BR, 1), 0)
        hs_ref[...] = jnp.where(row < N_NODES, h * dinv, 0.0)

    return pl.pallas_call(
        body,
        grid=(NP // _BR,),
        in_specs=[
            pl.BlockSpec((NC, _BR, D), lambda i: (0, i, 0)),
            pl.BlockSpec((_BR, D), lambda i: (i, 0)),
            pl.BlockSpec((NC, _BR, DEG_W), lambda i: (0, i, 0)),
            pl.BlockSpec((D, 2 * D), lambda i: (0, 0)),
            pl.BlockSpec((1, 2 * D), lambda i: (0, 0)),
            pl.BlockSpec((2 * D, D), lambda i: (0, 0)),
        ],
        out_specs=pl.BlockSpec((_BR, D), lambda i: (i, 0)),
        out_shape=jax.ShapeDtypeStruct((NP, D), jnp.float32),
    )(p, xs, degp, W1, b1, W2)


def _tc_final(q, hs, degp, b2):
    def body(q_ref, hs_ref, degp_ref, b2_ref, out_ref):
        dinv = _dinv_block(degp_ref[...])
        out_ref[...] = (q_ref[0] + q_ref[1] + hs_ref[...]) * dinv + b2_ref[...]

    return pl.pallas_call(
        body,
        grid=(NP // _BR,),
        in_specs=[
            pl.BlockSpec((NC, _BR, D), lambda i: (0, i, 0)),
            pl.BlockSpec((_BR, D), lambda i: (i, 0)),
            pl.BlockSpec((NC, _BR, DEG_W), lambda i: (0, i, 0)),
            pl.BlockSpec((1, D), lambda i: (0, 0)),
        ],
        out_specs=pl.BlockSpec((_BR, D), lambda i: (i, 0)),
        out_shape=jax.ShapeDtypeStruct((NP, D), jnp.float32),
    )(q, hs, degp, b2)


def kernel(sent_encoder_embeds, edge_index, W1, b1, W2, b2):
    x = sent_encoder_embeds
    src = edge_index[0]
    dst = edge_index[1]
    n_edges = src.shape[0]
    per_seg = NW * CHUNK * SEG
    ep = ((n_edges + per_seg - 1) // per_seg) * per_seg
    n_chunks = ep // (NW * CHUNK)
    pad = jnp.full((ep - n_edges,), N_NODES, jnp.int32)
    src_p = jnp.concatenate([src, pad])
    dst_p = jnp.concatenate([dst, pad])
    x_pad = jnp.pad(x, ((0, NP - N_NODES), (0, 0)))

    sc_agg = _make_sc_agg(n_chunks)
    degc = _sc_deg(dst_p, n_chunks)
    xs = _tc_prep(degc, x_pad)
    p = sc_agg(xs, src_p, dst_p)
    hs = _tc_mid(p, xs, degc, W1.astype(jnp.float32), b1.reshape(1, -1), W2.astype(jnp.float32))
    q = sc_agg(hs, src_p, dst_p)
    out = _tc_final(q, hs, degc, b2.reshape(1, -1))
    return out[:N_NODES]
